# baseline (speedup 1.0000x reference)
import functools

import jax
import jax.numpy as jnp
from jax import lax
from jax.experimental import pallas as pl
from jax.experimental.pallas import tpu as pltpu
from jax.experimental.pallas import tpu_sc as plsc

NUM_LATTICES = 1024
LATTICE_RANK = 4
NUM_INPUTS = 128
BATCH = 4096
LANES = 16

NUM_CORES = 2
NUM_SUBCORES = 16
NW = NUM_CORES * NUM_SUBCORES
TB = BATCH // NW
BGROUPS = TB // LANES
LCHUNK = 256
NCHUNK = NUM_LATTICES // LCHUNK
NGROUPS = NUM_LATTICES // LANES


def _tec_body(xtt_hbm, kt_hbm, idx_hbm, out_hbm, xv, kv, iv, outv):
    wid = lax.axis_index("s") * NUM_CORES + lax.axis_index("c")

    pltpu.sync_copy(xtt_hbm.at[wid], xv)
    pltpu.sync_copy(kt_hbm, kv)
    pltpu.sync_copy(idx_hbm, iv.at[pl.ds(0, NUM_LATTICES * LATTICE_RANK)])

    def clip_row(r):
        for j in range(NUM_INPUTS // LANES):
            v = xv[r, pl.ds(j * LANES, LANES)]
            xv[r, pl.ds(j * LANES, LANES)] = jnp.minimum(
                jnp.maximum(v, 0.0), 1.0)

    plsc.parallel_loop(0, NUM_INPUTS, unroll=2)(clip_row)

    def mobius_group(g):
        g16 = g * LANES
        cf = [kv[j, pl.ds(g16, LANES)] for j in range(16)]
        for dlev in (8, 4, 2, 1):
            for j in range(16):
                if j & dlev:
                    cf[j] = cf[j] - cf[j ^ dlev]
        for j in range(16):
            kv[j, pl.ds(g16, LANES)] = cf[j]

    plsc.parallel_loop(0, NGROUPS, unroll=1)(mobius_group)

    jvec = lax.broadcasted_iota(jnp.int32, (LANES,), 0)

    def splat(vec, j):
        return lax.gather(
            vec, jnp.full((LANES, 1), j, jnp.int32),
            lax.GatherDimensionNumbers(
                offset_dims=(), collapsed_slice_dims=(0,),
                start_index_map=(0,)),
            (1,), mode=lax.GatherScatterMode.PROMISE_IN_BOUNDS)

    for c in range(NCHUNK):
        def lat_body(ll, c=c):
            l = c * LCHUNK + ll
            lvec = jnp.full((LANES,), l, jnp.int32)
            cfl = plsc.load_gather(kv, [jvec, lvec])
            u = [splat(cfl, j) for j in range(16)]
            ivv = iv[pl.ds(l * LATTICE_RANK, LANES)]
            i0 = ivv[0]
            i1 = ivv[1]
            i2 = ivv[2]
            i3 = ivv[3]
            for gb in range(BGROUPS):
                bs = pl.ds(gb * LANES, LANES)
                x0 = xv[i0, bs]
                x1 = xv[i1, bs]
                x2 = xv[i2, bs]
                x3 = xv[i3, bs]
                tA = [u[j] + u[j + 8] * x0 for j in range(8)]
                tB = [tA[j] + tA[j + 4] * x1 for j in range(4)]
                tC = [tB[j] + tB[j + 2] * x2 for j in range(2)]
                outv[ll, bs] = tC[0] + tC[1] * x3

        plsc.parallel_loop(0, LCHUNK, unroll=2)(lat_body)
        pltpu.sync_copy(
            outv,
            out_hbm.at[pl.ds(c * LCHUNK, LCHUNK), pl.ds(wid * TB, TB)])


@functools.partial(jax.jit, static_argnames=())
def _rtl_sc(xtt, kt, idx):
    mesh = plsc.VectorSubcoreMesh(
        core_axis_name="c", subcore_axis_name="s")
    run = pl.kernel(
        _tec_body,
        out_type=jax.ShapeDtypeStruct((NUM_LATTICES, BATCH), jnp.float32),
        mesh=mesh,
        scratch_types=[
            pltpu.VMEM((NUM_INPUTS, TB), jnp.float32),
            pltpu.VMEM((LANES, NUM_LATTICES), jnp.float32),
            pltpu.VMEM((NUM_LATTICES * LATTICE_RANK + LANES,),
                       jnp.int32),
            pltpu.VMEM((LCHUNK, TB), jnp.float32),
        ],
        compiler_params=pltpu.CompilerParams(needs_layout_passes=False),
    )
    return run(xtt, kt, idx)


def kernel(x, lattice_indices, kernel):
    xtt = x.reshape(NW, TB, NUM_INPUTS).transpose(0, 2, 1)
    kt = kernel.T
    idx = lattice_indices.astype(jnp.int32).reshape(-1)
    return _rtl_sc(xtt, kt, idx).T


# speedup vs baseline: 1.2426x; 1.2426x over previous
import functools

import jax
import jax.numpy as jnp
from jax import lax
from jax.experimental import pallas as pl
from jax.experimental.pallas import tpu as pltpu
from jax.experimental.pallas import tpu_sc as plsc

NUM_LATTICES = 1024
LATTICE_RANK = 4
NUM_INPUTS = 128
BATCH = 4096
LANES = 16

NUM_CORES = 2
NUM_SUBCORES = 16
NW = NUM_CORES * NUM_SUBCORES
TB = BATCH // NW
BGROUPS = TB // LANES
LCHUNK = 256
NCHUNK = NUM_LATTICES // LCHUNK
NGROUPS = NUM_LATTICES // LANES


def _tec_body(xtt_hbm, kt_hbm, idx_hbm, out_hbm, xv, kv, iv, outvs, sems):
    wid = lax.axis_index("s") * NUM_CORES + lax.axis_index("c")

    pltpu.sync_copy(xtt_hbm.at[wid], xv)
    pltpu.sync_copy(kt_hbm, kv)
    pltpu.sync_copy(idx_hbm, iv.at[pl.ds(0, NUM_LATTICES * LATTICE_RANK)])

    def clip_row(r):
        for j in range(NUM_INPUTS // LANES):
            v = xv[r, pl.ds(j * LANES, LANES)]
            xv[r, pl.ds(j * LANES, LANES)] = jnp.minimum(
                jnp.maximum(v, 0.0), 1.0)

    plsc.parallel_loop(0, NUM_INPUTS, unroll=2)(clip_row)

    def mobius_group(g):
        g16 = g * LANES
        cf = [kv[j, pl.ds(g16, LANES)] for j in range(16)]
        for dlev in (8, 4, 2, 1):
            for j in range(16):
                if j & dlev:
                    cf[j] = cf[j] - cf[j ^ dlev]
        for j in range(16):
            kv[j, pl.ds(g16, LANES)] = cf[j]

    plsc.parallel_loop(0, NGROUPS, unroll=1)(mobius_group)

    jvec = lax.broadcasted_iota(jnp.int32, (LANES,), 0)

    def splat(vec, j):
        return lax.gather(
            vec, jnp.full((LANES, 1), j, jnp.int32),
            lax.GatherDimensionNumbers(
                offset_dims=(), collapsed_slice_dims=(0,),
                start_index_map=(0,)),
            (1,), mode=lax.GatherScatterMode.PROMISE_IN_BOUNDS)

    copies = [None, None]
    for c in range(NCHUNK):
        outv = outvs[c % 2]
        if copies[c % 2] is not None:
            copies[c % 2].wait()

        def lat_body(ll, c=c, outv=outv):
            l = c * LCHUNK + ll
            lvec = jnp.full((LANES,), l, jnp.int32)
            cfl = plsc.load_gather(kv, [jvec, lvec])
            u = [splat(cfl, j) for j in range(16)]
            ivv = iv[pl.ds(l * LATTICE_RANK, LANES)]
            i0 = ivv[0]
            i1 = ivv[1]
            i2 = ivv[2]
            i3 = ivv[3]
            for gb in range(BGROUPS):
                bs = pl.ds(gb * LANES, LANES)
                x0 = xv[i0, bs]
                x1 = xv[i1, bs]
                x2 = xv[i2, bs]
                x3 = xv[i3, bs]
                tA = [u[j] + u[j + 8] * x0 for j in range(8)]
                tB = [tA[j] + tA[j + 4] * x1 for j in range(4)]
                tC = [tB[j] + tB[j + 2] * x2 for j in range(2)]
                outv[ll, bs] = tC[0] + tC[1] * x3

        plsc.parallel_loop(0, LCHUNK, unroll=1)(lat_body)
        copies[c % 2] = pltpu.async_copy(
            outv,
            out_hbm.at[pl.ds(c * LCHUNK, LCHUNK), pl.ds(wid * TB, TB)],
            sems[c % 2])
    for cp in copies:
        cp.wait()


@functools.partial(jax.jit, static_argnames=())
def _rtl_sc(xtt, kt, idx):
    mesh = plsc.VectorSubcoreMesh(
        core_axis_name="c", subcore_axis_name="s")
    run = pl.kernel(
        _tec_body,
        out_type=jax.ShapeDtypeStruct((NUM_LATTICES, BATCH), jnp.float32),
        mesh=mesh,
        scratch_types=[
            pltpu.VMEM((NUM_INPUTS, TB), jnp.float32),
            pltpu.VMEM((LANES, NUM_LATTICES), jnp.float32),
            pltpu.VMEM((NUM_LATTICES * LATTICE_RANK + LANES,),
                       jnp.int32),
            [pltpu.VMEM((LCHUNK, TB), jnp.float32),
             pltpu.VMEM((LCHUNK, TB), jnp.float32)],
            [pltpu.SemaphoreType.DMA, pltpu.SemaphoreType.DMA],
        ],
        compiler_params=pltpu.CompilerParams(needs_layout_passes=False),
    )
    return run(xtt, kt, idx)


def kernel(x, lattice_indices, kernel):
    xtt = x.reshape(NW, TB, NUM_INPUTS).transpose(0, 2, 1)
    kt = kernel.T
    idx = lattice_indices.astype(jnp.int32).reshape(-1)
    return _rtl_sc(xtt, kt, idx).T


# speedup vs baseline: 1.2572x; 1.0117x over previous
import functools

import jax
import jax.numpy as jnp
from jax import lax
from jax.experimental import pallas as pl
from jax.experimental.pallas import tpu as pltpu
from jax.experimental.pallas import tpu_sc as plsc

NUM_LATTICES = 1024
LATTICE_RANK = 4
NUM_INPUTS = 128
BATCH = 4096
LANES = 16

NUM_CORES = 2
NUM_SUBCORES = 16
NW = NUM_CORES * NUM_SUBCORES
BSPLIT = 16
LSPLIT = NW // BSPLIT
TB = BATCH // BSPLIT
BGROUPS = TB // LANES
LPW = NUM_LATTICES // LSPLIT
LCHUNK = 128
NCHUNK = LPW // LCHUNK
NGROUPS = LPW // LANES


def _tec_body(xtt_hbm, kt_hbm, idx_hbm, out_hbm, xv, kv, iv, outvs, sems):
    wid = lax.axis_index("s") * NUM_CORES + lax.axis_index("c")
    bw = wid // LSPLIT
    lw = wid % LSPLIT

    pltpu.sync_copy(xtt_hbm.at[bw], xv)
    pltpu.sync_copy(kt_hbm.at[:, pl.ds(lw * LPW, LPW)], kv)
    pltpu.sync_copy(
        idx_hbm.at[pl.ds(lw * LPW * LATTICE_RANK, LPW * LATTICE_RANK)],
        iv.at[pl.ds(0, LPW * LATTICE_RANK)])

    def clip_row(r):
        for j in range(TB // LANES):
            v = xv[r, pl.ds(j * LANES, LANES)]
            xv[r, pl.ds(j * LANES, LANES)] = jnp.minimum(
                jnp.maximum(v, 0.0), 1.0)

    plsc.parallel_loop(0, NUM_INPUTS, unroll=2)(clip_row)

    def mobius_group(g):
        g16 = g * LANES
        cf = [kv[j, pl.ds(g16, LANES)] for j in range(16)]
        for dlev in (8, 4, 2, 1):
            for j in range(16):
                if j & dlev:
                    cf[j] = cf[j] - cf[j ^ dlev]
        for j in range(16):
            kv[j, pl.ds(g16, LANES)] = cf[j]

    plsc.parallel_loop(0, NGROUPS, unroll=1)(mobius_group)

    jvec = lax.broadcasted_iota(jnp.int32, (LANES,), 0)

    def splat(vec, j):
        return lax.gather(
            vec, jnp.full((LANES, 1), j, jnp.int32),
            lax.GatherDimensionNumbers(
                offset_dims=(), collapsed_slice_dims=(0,),
                start_index_map=(0,)),
            (1,), mode=lax.GatherScatterMode.PROMISE_IN_BOUNDS)

    copies = [None, None]
    for c in range(NCHUNK):
        outv = outvs[c % 2]
        if copies[c % 2] is not None:
            copies[c % 2].wait()

        def lat_body(ll, c=c, outv=outv):
            l = c * LCHUNK + ll
            lvec = jnp.full((LANES,), l, jnp.int32)
            cfl = plsc.load_gather(kv, [jvec, lvec])
            u = [splat(cfl, j) for j in range(16)]
            ivv = iv[pl.ds(l * LATTICE_RANK, LANES)]
            i0 = ivv[0]
            i1 = ivv[1]
            i2 = ivv[2]
            i3 = ivv[3]
            for gb in range(BGROUPS):
                bs = pl.ds(gb * LANES, LANES)
                x0 = xv[i0, bs]
                x1 = xv[i1, bs]
                x2 = xv[i2, bs]
                x3 = xv[i3, bs]
                tA = [u[j] + u[j + 8] * x0 for j in range(8)]
                tB = [tA[j] + tA[j + 4] * x1 for j in range(4)]
                tC = [tB[j] + tB[j + 2] * x2 for j in range(2)]
                outv[ll, bs] = tC[0] + tC[1] * x3

        plsc.parallel_loop(0, LCHUNK, unroll=1)(lat_body)
        copies[c % 2] = pltpu.async_copy(
            outv,
            out_hbm.at[pl.ds(lw * LPW + c * LCHUNK, LCHUNK),
                       pl.ds(bw * TB, TB)],
            sems[c % 2])
    for cp in copies:
        cp.wait()


@functools.partial(jax.jit, static_argnames=())
def _rtl_sc(xtt, kt, idx):
    mesh = plsc.VectorSubcoreMesh(
        core_axis_name="c", subcore_axis_name="s")
    run = pl.kernel(
        _tec_body,
        out_type=jax.ShapeDtypeStruct((NUM_LATTICES, BATCH), jnp.float32),
        mesh=mesh,
        scratch_types=[
            pltpu.VMEM((NUM_INPUTS, TB), jnp.float32),
            pltpu.VMEM((LANES, LPW), jnp.float32),
            pltpu.VMEM((LPW * LATTICE_RANK + LANES,),
                       jnp.int32),
            [pltpu.VMEM((LCHUNK, TB), jnp.float32),
             pltpu.VMEM((LCHUNK, TB), jnp.float32)],
            [pltpu.SemaphoreType.DMA, pltpu.SemaphoreType.DMA],
        ],
        compiler_params=pltpu.CompilerParams(needs_layout_passes=False),
    )
    return run(xtt, kt, idx)


def kernel(x, lattice_indices, kernel):
    xtt = x.reshape(BSPLIT, TB, NUM_INPUTS).transpose(0, 2, 1)
    kt = kernel.T
    idx = lattice_indices.astype(jnp.int32).reshape(-1)
    return _rtl_sc(xtt, kt, idx).T


# speedup vs baseline: 1.6667x; 1.3258x over previous
import functools

import jax
import jax.numpy as jnp
from jax import lax
from jax.experimental import pallas as pl
from jax.experimental.pallas import tpu as pltpu
from jax.experimental.pallas import tpu_sc as plsc

NUM_LATTICES = 1024
LATTICE_RANK = 4
NUM_INPUTS = 128
BATCH = 4096
LANES = 16

F_SC = 512
F_TC = NUM_LATTICES - F_SC

NUM_CORES = 2
NUM_SUBCORES = 16
NW = NUM_CORES * NUM_SUBCORES
BSPLIT = 16
LSPLIT = NW // BSPLIT
TB = BATCH // BSPLIT
BGROUPS = TB // LANES
LPW = F_SC // LSPLIT
LCHUNK = 128
NCHUNK = LPW // LCHUNK
NGROUPS = LPW // LANES

TCB = 256


def _tec_body(xtt_hbm, kt_hbm, idx_hbm, out_hbm, xv, kv, iv, outvs, sems):
    wid = lax.axis_index("s") * NUM_CORES + lax.axis_index("c")
    bw = wid // LSPLIT
    lw = wid % LSPLIT

    pltpu.sync_copy(xtt_hbm.at[bw], xv)
    pltpu.sync_copy(kt_hbm.at[:, pl.ds(lw * LPW, LPW)], kv)
    pltpu.sync_copy(
        idx_hbm.at[pl.ds(lw * LPW * LATTICE_RANK, LPW * LATTICE_RANK)],
        iv.at[pl.ds(0, LPW * LATTICE_RANK)])

    def clip_row(r):
        for j in range(TB // LANES):
            v = xv[r, pl.ds(j * LANES, LANES)]
            xv[r, pl.ds(j * LANES, LANES)] = jnp.minimum(
                jnp.maximum(v, 0.0), 1.0)

    plsc.parallel_loop(0, NUM_INPUTS, unroll=2)(clip_row)

    def mobius_group(g):
        g16 = g * LANES
        cf = [kv[j, pl.ds(g16, LANES)] for j in range(16)]
        for dlev in (8, 4, 2, 1):
            for j in range(16):
                if j & dlev:
                    cf[j] = cf[j] - cf[j ^ dlev]
        for j in range(16):
            kv[j, pl.ds(g16, LANES)] = cf[j]

    plsc.parallel_loop(0, NGROUPS, unroll=1)(mobius_group)

    jvec = lax.broadcasted_iota(jnp.int32, (LANES,), 0)

    def splat(vec, j):
        return lax.gather(
            vec, jnp.full((LANES, 1), j, jnp.int32),
            lax.GatherDimensionNumbers(
                offset_dims=(), collapsed_slice_dims=(0,),
                start_index_map=(0,)),
            (1,), mode=lax.GatherScatterMode.PROMISE_IN_BOUNDS)

    copies = [None, None]
    for c in range(NCHUNK):
        outv = outvs[c % 2]
        if copies[c % 2] is not None:
            copies[c % 2].wait()

        def lat_body(ll, c=c, outv=outv):
            l = c * LCHUNK + ll
            lvec = jnp.full((LANES,), l, jnp.int32)
            cfl = plsc.load_gather(kv, [jvec, lvec])
            u = [splat(cfl, j) for j in range(16)]
            ivv = iv[pl.ds(l * LATTICE_RANK, LANES)]
            i0 = ivv[0]
            i1 = ivv[1]
            i2 = ivv[2]
            i3 = ivv[3]
            for gb in range(BGROUPS):
                bs = pl.ds(gb * LANES, LANES)
                x0 = xv[i0, bs]
                x1 = xv[i1, bs]
                x2 = xv[i2, bs]
                x3 = xv[i3, bs]
                tA = [u[j] + u[j + 8] * x0 for j in range(8)]
                tB = [tA[j] + tA[j + 4] * x1 for j in range(4)]
                tC = [tB[j] + tB[j + 2] * x2 for j in range(2)]
                outv[ll, bs] = tC[0] + tC[1] * x3

        plsc.parallel_loop(0, LCHUNK, unroll=1)(lat_body)
        copies[c % 2] = pltpu.async_copy(
            outv,
            out_hbm.at[pl.ds(lw * LPW + c * LCHUNK, LCHUNK),
                       pl.ds(bw * TB, TB)],
            sems[c % 2])
    for cp in copies:
        cp.wait()


@functools.partial(jax.jit, static_argnames=())
def _rtl_sc(xtt, kt, idx):
    mesh = plsc.VectorSubcoreMesh(
        core_axis_name="c", subcore_axis_name="s")
    run = pl.kernel(
        _tec_body,
        out_type=jax.ShapeDtypeStruct((F_SC, BATCH), jnp.float32),
        mesh=mesh,
        scratch_types=[
            pltpu.VMEM((NUM_INPUTS, TB), jnp.float32),
            pltpu.VMEM((LANES, LPW), jnp.float32),
            pltpu.VMEM((LPW * LATTICE_RANK + LANES,),
                       jnp.int32),
            [pltpu.VMEM((LCHUNK, TB), jnp.float32),
             pltpu.VMEM((LCHUNK, TB), jnp.float32)],
            [pltpu.SemaphoreType.DMA, pltpu.SemaphoreType.DMA],
        ],
        compiler_params=pltpu.CompilerParams(needs_layout_passes=False),
    )
    return run(xtt, kt, idx)


def _tc_tile(x_ref, idxt_ref, ktt_ref, out_ref):
    x = jnp.minimum(jnp.maximum(x_ref[...], 0.0), 1.0)
    col = lax.broadcasted_iota(jnp.int32, (NUM_INPUTS, F_TC), 0)
    xg = []
    for d in range(LATTICE_RANK):
        onehot = jnp.where(col == idxt_ref[d][None, :], 1.0, 0.0)
        xg.append(lax.dot_general(
            x, onehot, (((1,), (0,)), ((), ())),
            preferred_element_type=jnp.float32))
    cf = [ktt_ref[j][None, :] for j in range(16)]
    for dlev in (8, 4, 2, 1):
        for j in range(16):
            if j & dlev:
                cf[j] = cf[j] - cf[j ^ dlev]
    tA = [cf[j] + cf[j + 8] * xg[0] for j in range(8)]
    tB = [tA[j] + tA[j + 4] * xg[1] for j in range(4)]
    tC = [tB[j] + tB[j + 2] * xg[2] for j in range(2)]
    out_ref[...] = tC[0] + tC[1] * xg[3]


@functools.partial(jax.jit, static_argnames=())
def _rtl_tc(x, idxt_tc, ktt_tc):
    return pl.pallas_call(
        _tc_tile,
        grid=(BATCH // TCB,),
        in_specs=[
            pl.BlockSpec((TCB, NUM_INPUTS), lambda i: (i, 0)),
            pl.BlockSpec((LATTICE_RANK, F_TC), lambda i: (0, 0)),
            pl.BlockSpec((16, F_TC), lambda i: (0, 0)),
        ],
        out_specs=pl.BlockSpec((TCB, F_TC), lambda i: (i, 0)),
        out_shape=jax.ShapeDtypeStruct((BATCH, F_TC), jnp.float32),
    )(x, idxt_tc, ktt_tc)


def kernel(x, lattice_indices, kernel):
    xtt = x.reshape(BSPLIT, TB, NUM_INPUTS).transpose(0, 2, 1)
    idx = lattice_indices.astype(jnp.int32)
    kt_sc = kernel[:F_SC].T
    idx_sc = idx[:F_SC].reshape(-1)
    out_sc_t = _rtl_sc(xtt, kt_sc, idx_sc)
    out_tc = _rtl_tc(x, idx[F_SC:].T, kernel[F_SC:].T)
    return jnp.concatenate([out_sc_t.T, out_tc], axis=1)


# speedup vs baseline: 1.6692x; 1.0015x over previous
import functools

import jax
import jax.numpy as jnp
from jax import lax
from jax.experimental import pallas as pl
from jax.experimental.pallas import tpu as pltpu
from jax.experimental.pallas import tpu_sc as plsc

NUM_LATTICES = 1024
LATTICE_RANK = 4
NUM_INPUTS = 128
BATCH = 4096
LANES = 16

F_SC = 512
F_TC = NUM_LATTICES - F_SC

NUM_CORES = 2
NUM_SUBCORES = 16
NW = NUM_CORES * NUM_SUBCORES
BSPLIT = 16
LSPLIT = NW // BSPLIT
TB = BATCH // BSPLIT
BGROUPS = TB // LANES
LPW = F_SC // LSPLIT
LCHUNK = 128
NCHUNK = LPW // LCHUNK
NGROUPS = LPW // LANES

TCB = 256


def _tec_body(xtt_hbm, kt_hbm, idx_hbm, out_hbm, xv, kv, iv, outvs, sems):
    wid = lax.axis_index("s") * NUM_CORES + lax.axis_index("c")
    bw = wid // LSPLIT
    lw = wid % LSPLIT

    pltpu.sync_copy(xtt_hbm.at[bw], xv)
    pltpu.sync_copy(kt_hbm.at[:, pl.ds(lw * LPW, LPW)], kv)
    pltpu.sync_copy(
        idx_hbm.at[pl.ds(lw * LPW * LATTICE_RANK, LPW * LATTICE_RANK)],
        iv.at[pl.ds(0, LPW * LATTICE_RANK)])

    def clip_row(r):
        for j in range(TB // LANES):
            v = xv[r, pl.ds(j * LANES, LANES)]
            xv[r, pl.ds(j * LANES, LANES)] = jnp.minimum(
                jnp.maximum(v, 0.0), 1.0)

    plsc.parallel_loop(0, NUM_INPUTS, unroll=2)(clip_row)

    def mobius_group(g):
        g16 = g * LANES
        cf = [kv[j, pl.ds(g16, LANES)] for j in range(16)]
        for dlev in (8, 4, 2, 1):
            for j in range(16):
                if j & dlev:
                    cf[j] = cf[j] - cf[j ^ dlev]
        for j in range(16):
            kv[j, pl.ds(g16, LANES)] = cf[j]

    plsc.parallel_loop(0, NGROUPS, unroll=1)(mobius_group)

    jvec = lax.broadcasted_iota(jnp.int32, (LANES,), 0)

    def splat(vec, j):
        return lax.gather(
            vec, jnp.full((LANES, 1), j, jnp.int32),
            lax.GatherDimensionNumbers(
                offset_dims=(), collapsed_slice_dims=(0,),
                start_index_map=(0,)),
            (1,), mode=lax.GatherScatterMode.PROMISE_IN_BOUNDS)

    copies = [None, None]
    for c in range(NCHUNK):
        outv = outvs[c % 2]
        if copies[c % 2] is not None:
            copies[c % 2].wait()

        def lat_body(ll, c=c, outv=outv):
            l = c * LCHUNK + ll
            lvec = jnp.full((LANES,), l, jnp.int32)
            cfl = plsc.load_gather(kv, [jvec, lvec])
            u = [splat(cfl, j) for j in range(16)]
            ivv = iv[pl.ds(l * LATTICE_RANK, LANES)]
            i0 = ivv[0]
            i1 = ivv[1]
            i2 = ivv[2]
            i3 = ivv[3]
            for gb in range(BGROUPS):
                bs = pl.ds(gb * LANES, LANES)
                x0 = xv[i0, bs]
                x1 = xv[i1, bs]
                x2 = xv[i2, bs]
                x3 = xv[i3, bs]
                tA = [u[j] + u[j + 8] * x0 for j in range(8)]
                tB = [tA[j] + tA[j + 4] * x1 for j in range(4)]
                tC = [tB[j] + tB[j + 2] * x2 for j in range(2)]
                outv[ll, bs] = tC[0] + tC[1] * x3

        plsc.parallel_loop(0, LCHUNK, unroll=1)(lat_body)
        copies[c % 2] = pltpu.async_copy(
            outv,
            out_hbm.at[pl.ds(lw * LPW + c * LCHUNK, LCHUNK),
                       pl.ds(bw * TB, TB)],
            sems[c % 2])
    for cp in copies:
        cp.wait()


@functools.partial(jax.jit, static_argnames=())
def _rtl_sc(xtt, kt, idx):
    mesh = plsc.VectorSubcoreMesh(
        core_axis_name="c", subcore_axis_name="s")
    run = pl.kernel(
        _tec_body,
        out_type=jax.ShapeDtypeStruct((F_SC, BATCH), jnp.float32),
        mesh=mesh,
        scratch_types=[
            pltpu.VMEM((NUM_INPUTS, TB), jnp.float32),
            pltpu.VMEM((LANES, LPW), jnp.float32),
            pltpu.VMEM((LPW * LATTICE_RANK + LANES,),
                       jnp.int32),
            [pltpu.VMEM((LCHUNK, TB), jnp.float32),
             pltpu.VMEM((LCHUNK, TB), jnp.float32)],
            [pltpu.SemaphoreType.DMA, pltpu.SemaphoreType.DMA],
        ],
        compiler_params=pltpu.CompilerParams(needs_layout_passes=False),
    )
    return run(xtt, kt, idx)


def _tc_tile(x_ref, idxt_ref, ktt_ref, out_ref):
    x = jnp.minimum(jnp.maximum(x_ref[...], 0.0), 1.0)
    col = lax.broadcasted_iota(jnp.int32, (NUM_INPUTS, F_TC), 0)
    xg = []
    for d in range(LATTICE_RANK):
        onehot = jnp.where(col == idxt_ref[d][None, :], 1.0, 0.0)
        xg.append(lax.dot_general(
            x, onehot, (((1,), (0,)), ((), ())),
            preferred_element_type=jnp.float32,
            precision=lax.Precision.HIGHEST))
    cf = [ktt_ref[j][None, :] for j in range(16)]
    for dlev in (8, 4, 2, 1):
        for j in range(16):
            if j & dlev:
                cf[j] = cf[j] - cf[j ^ dlev]
    tA = [cf[j] + cf[j + 8] * xg[0] for j in range(8)]
    tB = [tA[j] + tA[j + 4] * xg[1] for j in range(4)]
    tC = [tB[j] + tB[j + 2] * xg[2] for j in range(2)]
    out_ref[...] = tC[0] + tC[1] * xg[3]


@functools.partial(jax.jit, static_argnames=())
def _rtl_tc(x, idxt_tc, ktt_tc):
    return pl.pallas_call(
        _tc_tile,
        grid=(BATCH // TCB,),
        in_specs=[
            pl.BlockSpec((TCB, NUM_INPUTS), lambda i: (i, 0)),
            pl.BlockSpec((LATTICE_RANK, F_TC), lambda i: (0, 0)),
            pl.BlockSpec((16, F_TC), lambda i: (0, 0)),
        ],
        out_specs=pl.BlockSpec((TCB, F_TC), lambda i: (i, 0)),
        out_shape=jax.ShapeDtypeStruct((BATCH, F_TC), jnp.float32),
    )(x, idxt_tc, ktt_tc)


def kernel(x, lattice_indices, kernel):
    xtt = x.reshape(BSPLIT, TB, NUM_INPUTS).transpose(0, 2, 1)
    idx = lattice_indices.astype(jnp.int32)
    kt_sc = kernel[:F_SC].T
    idx_sc = idx[:F_SC].reshape(-1)
    out_sc_t = _rtl_sc(xtt, kt_sc, idx_sc)
    out_tc = _rtl_tc(x, idx[F_SC:].T, kernel[F_SC:].T)
    return jnp.concatenate([out_sc_t.T, out_tc], axis=1)


# speedup vs baseline: 1.6700x; 1.0005x over previous
import functools

import jax
import jax.numpy as jnp
from jax import lax
from jax.experimental import pallas as pl
from jax.experimental.pallas import tpu as pltpu
from jax.experimental.pallas import tpu_sc as plsc

NUM_LATTICES = 1024
LATTICE_RANK = 4
NUM_INPUTS = 128
BATCH = 4096
LANES = 16

F_SC = 256
F_TC = NUM_LATTICES - F_SC

NUM_CORES = 2
NUM_SUBCORES = 16
NW = NUM_CORES * NUM_SUBCORES
BSPLIT = 16
LSPLIT = NW // BSPLIT
TB = BATCH // BSPLIT
BGROUPS = TB // LANES
LPW = F_SC // LSPLIT
LCHUNK = LPW // 2
NCHUNK = LPW // LCHUNK
NGROUPS = LPW // LANES

TCB = 256


def _tec_body(xtt_hbm, kt_hbm, idx_hbm, out_hbm, xv, kv, iv, outvs, sems):
    wid = lax.axis_index("s") * NUM_CORES + lax.axis_index("c")
    bw = wid // LSPLIT
    lw = wid % LSPLIT

    pltpu.sync_copy(xtt_hbm.at[bw], xv)
    pltpu.sync_copy(kt_hbm.at[:, pl.ds(lw * LPW, LPW)], kv)
    pltpu.sync_copy(
        idx_hbm.at[pl.ds(lw * LPW * LATTICE_RANK, LPW * LATTICE_RANK)],
        iv.at[pl.ds(0, LPW * LATTICE_RANK)])

    def clip_row(r):
        for j in range(TB // LANES):
            v = xv[r, pl.ds(j * LANES, LANES)]
            xv[r, pl.ds(j * LANES, LANES)] = jnp.minimum(
                jnp.maximum(v, 0.0), 1.0)

    plsc.parallel_loop(0, NUM_INPUTS, unroll=2)(clip_row)

    def mobius_group(g):
        g16 = g * LANES
        cf = [kv[j, pl.ds(g16, LANES)] for j in range(16)]
        for dlev in (8, 4, 2, 1):
            for j in range(16):
                if j & dlev:
                    cf[j] = cf[j] - cf[j ^ dlev]
        for j in range(16):
            kv[j, pl.ds(g16, LANES)] = cf[j]

    plsc.parallel_loop(0, NGROUPS, unroll=1)(mobius_group)

    jvec = lax.broadcasted_iota(jnp.int32, (LANES,), 0)

    def splat(vec, j):
        return lax.gather(
            vec, jnp.full((LANES, 1), j, jnp.int32),
            lax.GatherDimensionNumbers(
                offset_dims=(), collapsed_slice_dims=(0,),
                start_index_map=(0,)),
            (1,), mode=lax.GatherScatterMode.PROMISE_IN_BOUNDS)

    copies = [None, None]
    for c in range(NCHUNK):
        outv = outvs[c % 2]
        if copies[c % 2] is not None:
            copies[c % 2].wait()

        def lat_body(ll, c=c, outv=outv):
            l = c * LCHUNK + ll
            lvec = jnp.full((LANES,), l, jnp.int32)
            cfl = plsc.load_gather(kv, [jvec, lvec])
            u = [splat(cfl, j) for j in range(16)]
            ivv = iv[pl.ds(l * LATTICE_RANK, LANES)]
            i0 = ivv[0]
            i1 = ivv[1]
            i2 = ivv[2]
            i3 = ivv[3]
            for gb in range(BGROUPS):
                bs = pl.ds(gb * LANES, LANES)
                x0 = xv[i0, bs]
                x1 = xv[i1, bs]
                x2 = xv[i2, bs]
                x3 = xv[i3, bs]
                tA = [u[j] + u[j + 8] * x0 for j in range(8)]
                tB = [tA[j] + tA[j + 4] * x1 for j in range(4)]
                tC = [tB[j] + tB[j + 2] * x2 for j in range(2)]
                outv[ll, bs] = tC[0] + tC[1] * x3

        plsc.parallel_loop(0, LCHUNK, unroll=1)(lat_body)
        copies[c % 2] = pltpu.async_copy(
            outv,
            out_hbm.at[pl.ds(lw * LPW + c * LCHUNK, LCHUNK),
                       pl.ds(bw * TB, TB)],
            sems[c % 2])
    for cp in copies:
        cp.wait()


@functools.partial(jax.jit, static_argnames=())
def _rtl_sc(xtt, kt, idx):
    mesh = plsc.VectorSubcoreMesh(
        core_axis_name="c", subcore_axis_name="s")
    run = pl.kernel(
        _tec_body,
        out_type=jax.ShapeDtypeStruct((F_SC, BATCH), jnp.float32),
        mesh=mesh,
        scratch_types=[
            pltpu.VMEM((NUM_INPUTS, TB), jnp.float32),
            pltpu.VMEM((LANES, LPW), jnp.float32),
            pltpu.VMEM((LPW * LATTICE_RANK + LANES,),
                       jnp.int32),
            [pltpu.VMEM((LCHUNK, TB), jnp.float32),
             pltpu.VMEM((LCHUNK, TB), jnp.float32)],
            [pltpu.SemaphoreType.DMA, pltpu.SemaphoreType.DMA],
        ],
        compiler_params=pltpu.CompilerParams(needs_layout_passes=False),
    )
    return run(xtt, kt, idx)


def _tc_tile(x_ref, idxt_ref, ktt_ref, out_ref):
    x = jnp.minimum(jnp.maximum(x_ref[...], 0.0), 1.0)
    col = lax.broadcasted_iota(jnp.int32, (NUM_INPUTS, F_TC), 0)
    xg = []
    for d in range(LATTICE_RANK):
        onehot = jnp.where(col == idxt_ref[d][None, :], 1.0, 0.0)
        xg.append(lax.dot_general(
            x, onehot, (((1,), (0,)), ((), ())),
            preferred_element_type=jnp.float32,
            precision=lax.Precision.HIGHEST))
    cf = [ktt_ref[j][None, :] for j in range(16)]
    for dlev in (8, 4, 2, 1):
        for j in range(16):
            if j & dlev:
                cf[j] = cf[j] - cf[j ^ dlev]
    tA = [cf[j] + cf[j + 8] * xg[0] for j in range(8)]
    tB = [tA[j] + tA[j + 4] * xg[1] for j in range(4)]
    tC = [tB[j] + tB[j + 2] * xg[2] for j in range(2)]
    out_ref[...] = tC[0] + tC[1] * xg[3]


@functools.partial(jax.jit, static_argnames=())
def _rtl_tc(x, idxt_tc, ktt_tc):
    return pl.pallas_call(
        _tc_tile,
        grid=(BATCH // TCB,),
        in_specs=[
            pl.BlockSpec((TCB, NUM_INPUTS), lambda i: (i, 0)),
            pl.BlockSpec((LATTICE_RANK, F_TC), lambda i: (0, 0)),
            pl.BlockSpec((16, F_TC), lambda i: (0, 0)),
        ],
        out_specs=pl.BlockSpec((TCB, F_TC), lambda i: (i, 0)),
        out_shape=jax.ShapeDtypeStruct((BATCH, F_TC), jnp.float32),
    )(x, idxt_tc, ktt_tc)


def kernel(x, lattice_indices, kernel):
    xtt = x.reshape(BSPLIT, TB, NUM_INPUTS).transpose(0, 2, 1)
    idx = lattice_indices.astype(jnp.int32)
    kt_sc = kernel[:F_SC].T
    idx_sc = idx[:F_SC].reshape(-1)
    out_sc_t = _rtl_sc(xtt, kt_sc, idx_sc)
    out_tc = _rtl_tc(x, idx[F_SC:].T, kernel[F_SC:].T)
    return jnp.concatenate([out_sc_t.T, out_tc], axis=1)
